# baseline (device time: 38800 ns/iter reference)
import jax
import jax.numpy as jnp
from jax import lax
from jax.experimental import pallas as pl
from jax.experimental.pallas import tpu as pltpu

N_DEV = 16
BF16 = jnp.bfloat16
F32 = jnp.float32

import os as _os
if _os.environ.get("KERNEL_DEBUG_MESH"):
    import sys as _sys
    from pathlib import Path as _Path
    _sys.path.insert(0, str(_Path(__file__).parent))
    import distributed_mesh_v7x as _dm
    _mesh = _dm.get_mesh("i", world_size=16)
    for _i, _d in enumerate(_mesh.devices.flat):
        print("MESHMAP", _i, _d.coords, getattr(_d, "core_on_chip", None),
              file=_sys.stderr)

M, N = 2048, 512
W = N // 2

ROWS = [256, 256, 256, 256, 256, 256, 256, 128,
        128, 256, 128, 128, 256, 128, 128, 256, 128, 128, 256]
OFFS = [sum(ROWS[:k]) for k in range(len(ROWS))]
TOT = sum(ROWS)
FWD = {16: 10, 17: 11, 18: 12}

SCHED = [
    ("s", 0, 1), ("s", 1, 1), ("s", 0, 3), ("s", 1, 3),
    ("s", 0, 0), ("s", 1, 0), ("s", 0, 2), ("s", 1, 2),
    ("f", 0, 1), ("s", 0, 4), ("f", 1, 1), ("s", 1, 4),
    ("f", 0, 3), ("s", 0, 5), ("f", 1, 3), ("s", 1, 5),
    ("f", 0, 0), ("f", 0, 4), ("s", 0, 6),
    ("f", 1, 0), ("f", 1, 4), ("s", 1, 6),
    ("f", 0, 2), ("f", 0, 5), ("f", 1, 2), ("f", 1, 5),
    ("f", 0, 6), ("s", 0, 7), ("f", 1, 6), ("s", 1, 7),
    ("f", 0, 7), ("s", 0, 8), ("s", 0, 10), ("s", 0, 13),
    ("f", 1, 7), ("s", 1, 8), ("s", 1, 10), ("s", 1, 13),
    ("f", 0, 8), ("s", 0, 9), ("s", 0, 11), ("s", 0, 14),
    ("f", 1, 8), ("s", 1, 9), ("s", 1, 11), ("s", 1, 14),
    ("f", 0, 9), ("s", 0, 12), ("s", 0, 15),
    ("f", 1, 9), ("s", 1, 12), ("s", 1, 15),
    ("w", 0, 10), ("s", 0, 16), ("p", 0, 10),
    ("w", 1, 10), ("s", 1, 16), ("p", 1, 10),
    ("w", 0, 11), ("s", 0, 17), ("p", 0, 11),
    ("w", 1, 11), ("s", 1, 17), ("p", 1, 11),
    ("w", 0, 12), ("s", 0, 18), ("p", 0, 12),
    ("w", 1, 12), ("s", 1, 18), ("p", 1, 12),
    ("f", 0, 13), ("f", 0, 14), ("f", 0, 15),
    ("f", 0, 16), ("f", 0, 17), ("f", 0, 18),
    ("f", 1, 13), ("f", 1, 14), ("f", 1, 15),
    ("f", 1, 16), ("f", 1, 17), ("f", 1, 18),
]


def kernel(x):
    def body(x_ref, out_ref, sb_a, rb_a, sb_b, rb_b, sems):
        me = lax.axis_index("i")
        p = lax.rem(me, 4)
        z = lax.div(me, 4)
        xb = jnp.where((p == 1) | (p == 2), 1, 0)
        yb = jnp.where(p >= 2, 1, 0)

        x_partner = z * 4 + jnp.where(lax.rem(p, 2) == 0, p + 1, p - 1)
        y_partner = z * 4 + (3 - p)
        zbit0 = lax.rem(z, 2)
        zbit1 = lax.div(z, 2)
        zp1 = (z + 1 - 2 * zbit0) * 4 + p
        zp2 = (z + 2 - 4 * zbit1) * 4 + p

        barrier_sem = pltpu.get_barrier_semaphore()
        for nbr in [x_partner, y_partner, zp1, zp2]:
            pl.semaphore_signal(
                barrier_sem,
                inc=1,
                device_id=(nbr,),
                device_id_type=pl.DeviceIdType.MESH,
            )
        pl.semaphore_wait(barrier_sem, 4)

        def half_units(c0, sel1, sel2, pn1, pn2, zorder, sb):
            mine_1 = sel1 * 1024
            theirs_1 = (1 - sel1) * 1024
            mine_2 = mine_1 + sel2 * 512
            theirs_2 = mine_1 + (1 - sel2) * 512
            send_s1 = theirs_1 + (1 - sel2) * 512
            send_s2 = theirs_1 + sel2 * 512
            recv_s2 = theirs_1 + (1 - sel2) * 512

            (zb_a, zp_a), (zb_b, zp_b) = zorder
            z_keep1 = mine_2 + zb_a * 256
            z_send1 = mine_2 + (1 - zb_a) * 256
            z_keep2 = z_keep1 + zb_b * 128
            z_send2 = z_keep1 + (1 - zb_b) * 128
            koff2 = zb_a * 256 + zb_b * 128
            soff2 = zb_a * 256 + (1 - zb_b) * 128
            soff1 = (1 - zb_a) * 256
            sra_off = (1 - zb_a) * 256
            srb_off = zb_a * 256

            def col(ref, base, rows):
                return ref[pl.ds(base, rows), pl.ds(c0, W)]

            def setcol(ref, base, rows, val):
                ref[pl.ds(base, rows), pl.ds(c0, W)] = val

            def sbput(base, rows, val):
                sb[pl.ds(base, rows), :] = val

            units = []

            units.append((
                pn1,
                lambda b: b.__setitem__(
                    ..., col(x_ref, send_s2 + sra_off, 256).astype(BF16)
                ),
                lambda rb: setcol(
                    out_ref, z_send1, 256,
                    (col(x_ref, z_send1, 256)
                     + rb[...].astype(F32)).astype(BF16),
                ),
            ))
            units.append((
                pn1,
                lambda b: b.__setitem__(
                    ..., col(x_ref, send_s1 + sra_off, 256).astype(BF16)
                ),
                lambda rb: setcol(
                    out_ref, theirs_2 + sra_off, 256,
                    (col(x_ref, theirs_2 + sra_off, 256)
                     + rb[...].astype(F32)).astype(BF16),
                ),
            ))
            units.append((
                pn1,
                lambda b: b.__setitem__(
                    ..., col(x_ref, send_s2 + srb_off, 256).astype(BF16)
                ),
                lambda rb: setcol(
                    out_ref, z_keep1, 256,
                    (col(x_ref, z_keep1, 256)
                     + rb[...].astype(F32)).astype(BF16),
                ),
            ))
            units.append((
                pn1,
                lambda b: b.__setitem__(
                    ..., col(x_ref, send_s1 + srb_off, 256).astype(BF16)
                ),
                lambda rb: setcol(
                    out_ref, theirs_2 + srb_off, 256,
                    (col(x_ref, theirs_2 + srb_off, 256)
                     + rb[...].astype(F32)).astype(BF16),
                ),
            ))
            units.append((
                pn2,
                lambda b: b.__setitem__(
                    ..., col(out_ref, theirs_2 + sra_off, 256)
                ),
                lambda rb: setcol(
                    out_ref, z_send1, 256,
                    col(out_ref, z_send1, 256) + rb[...],
                ),
            ))
            units.append((
                pn2,
                lambda b: b.__setitem__(
                    ..., col(out_ref, theirs_2 + srb_off, 256)
                ),
                lambda rb: setcol(
                    out_ref, z_keep1, 256,
                    col(out_ref, z_keep1, 256) + rb[...],
                ),
            ))
            units.append((
                zp_a,
                lambda b: b.__setitem__(..., col(out_ref, z_send1, 256)),
                lambda rb: setcol(
                    out_ref, z_keep1, 256,
                    col(out_ref, z_keep1, 256) + rb[...],
                ),
            ))

            def proc7(rb):
                vb = col(out_ref, z_keep2, 128) + rb[...]
                setcol(out_ref, z_keep2, 128, vb)
                sbput(OFFS[8], 128, vb)
                sbput(OFFS[9] + zb_b * 128, 128, vb)
                sbput(OFFS[10], 128, vb)
                sbput(OFFS[13], 128, vb)

            units.append((
                zp_b,
                lambda b: b.__setitem__(..., col(out_ref, z_send2, 128)),
                proc7,
            ))

            def proc8(rb):
                rbv = rb[...]
                setcol(out_ref, z_send2, 128, rbv)
                sbput(OFFS[9] + (1 - zb_b) * 128, 128, rbv)
                sbput(OFFS[11], 128, rbv)
                sbput(OFFS[14], 128, rbv)

            units.append((zp_b, None, proc8))

            def proc9(rb):
                rbv = rb[...]
                setcol(out_ref, z_send1, 256, rbv)
                sbput(OFFS[12], 256, rbv)
                sbput(OFFS[15], 256, rbv)

            units.append((zp_a, None, proc9))

            units.append((
                pn2,
                None,
                lambda rb: setcol(out_ref, theirs_2 + koff2, 128, rb[...]),
            ))
            units.append((
                pn2,
                None,
                lambda rb: setcol(out_ref, theirs_2 + soff2, 128, rb[...]),
            ))
            units.append((
                pn2,
                None,
                lambda rb: setcol(out_ref, theirs_2 + soff1, 256, rb[...]),
            ))
            units.append((
                pn1,
                None,
                lambda rb: setcol(out_ref, send_s2 + koff2, 128, rb[...]),
            ))
            units.append((
                pn1,
                None,
                lambda rb: setcol(out_ref, send_s2 + soff2, 128, rb[...]),
            ))
            units.append((
                pn1,
                None,
                lambda rb: setcol(out_ref, send_s2 + soff1, 256, rb[...]),
            ))
            units.append((
                pn1,
                None,
                lambda rb: setcol(out_ref, recv_s2 + koff2, 128, rb[...]),
            ))
            units.append((
                pn1,
                None,
                lambda rb: setcol(out_ref, recv_s2 + soff2, 128, rb[...]),
            ))
            units.append((
                pn1,
                None,
                lambda rb: setcol(out_ref, recv_s2 + soff1, 256, rb[...]),
            ))
            return units

        units_a = half_units(
            0, xb, yb, x_partner, y_partner,
            ((zbit0, zp1), (zbit1, zp2)), sb_a,
        )
        units_b = half_units(
            W, yb, xb, y_partner, x_partner,
            ((zbit1, zp2), (zbit0, zp1)), sb_b,
        )

        halves = [
            (units_a, sb_a, rb_a, 0, 1),
            (units_b, sb_b, rb_b, 2, 3),
        ]

        def make_rdma(h, k):
            units, sb, rb, srow, rrow = halves[h]
            partner = units[k][0]
            if k in FWD:
                j = FWD[k]
                src = rb.at[pl.ds(OFFS[j], ROWS[j]), :]
            else:
                src = sb.at[pl.ds(OFFS[k], ROWS[k]), :]
            return pltpu.make_async_remote_copy(
                src_ref=src,
                dst_ref=rb.at[pl.ds(OFFS[k], ROWS[k]), :],
                send_sem=sems.at[srow, k],
                recv_sem=sems.at[rrow, k],
                device_id=(partner,),
                device_id_type=pl.DeviceIdType.MESH,
            )

        for op, h, k in SCHED:
            units, sb, rb, _, _ = halves[h]
            _, prep, proc = units[k]
            if op == "s":
                if prep is not None:
                    prep(sb.at[pl.ds(OFFS[k], ROWS[k]), :])
                make_rdma(h, k).start()
            elif op == "w":
                make_rdma(h, k).wait_recv()
            elif op == "p":
                proc(rb.at[pl.ds(OFFS[k], ROWS[k]), :])
            else:
                make_rdma(h, k).wait_recv()
                proc(rb.at[pl.ds(OFFS[k], ROWS[k]), :])

        started = sorted({(h2, k2) for op2, h2, k2 in SCHED if op2 == "s"})
        recvd = {(h2, k2) for op2, h2, k2 in SCHED if op2 in ("f", "w")}
        for h2, k2 in started:
            if (h2, k2) not in recvd:
                make_rdma(h2, k2).wait_recv()
        for h2, k2 in started:
            make_rdma(h2, k2).wait_send()

    out_shape = jax.ShapeDtypeStruct((M, N), BF16)
    return pl.pallas_call(
        body,
        out_shape=out_shape,
        in_specs=[pl.BlockSpec(memory_space=pltpu.VMEM)],
        out_specs=pl.BlockSpec(memory_space=pltpu.VMEM),
        scratch_shapes=[
            pltpu.VMEM((TOT, W), BF16),
            pltpu.VMEM((TOT, W), BF16),
            pltpu.VMEM((TOT, W), BF16),
            pltpu.VMEM((TOT, W), BF16),
            pltpu.SemaphoreType.DMA((4, len(ROWS))),
        ],
        compiler_params=pltpu.CompilerParams(collective_id=0),
    )(x)


# device time: 38712 ns/iter; 1.0023x vs baseline; 1.0023x over previous
import jax
import jax.numpy as jnp
from jax import lax
from jax.experimental import pallas as pl
from jax.experimental.pallas import tpu as pltpu

N_DEV = 16
BF16 = jnp.bfloat16
F32 = jnp.float32

import os as _os
if _os.environ.get("KERNEL_DEBUG_MESH"):
    import sys as _sys
    from pathlib import Path as _Path
    _sys.path.insert(0, str(_Path(__file__).parent))
    import distributed_mesh_v7x as _dm
    _mesh = _dm.get_mesh("i", world_size=16)
    for _i, _d in enumerate(_mesh.devices.flat):
        print("MESHMAP", _i, _d.coords, getattr(_d, "core_on_chip", None),
              file=_sys.stderr)

M, N = 2048, 512
W = N // 2

ROWS = [512, 512, 256, 256, 256, 128, 128, 256,
        128, 128, 256, 128, 128, 256, 128, 128, 256]
OFFS = [sum(ROWS[:k]) for k in range(len(ROWS))]
TOT = sum(ROWS)
FWD = {14: 8, 15: 9, 16: 10}

SCHED = [
    ("s", 0, 0), ("s", 1, 0), ("s", 0, 1), ("s", 1, 1),
    ("f", 0, 0), ("s", 0, 2), ("s", 0, 3),
    ("f", 1, 0), ("s", 1, 2), ("s", 1, 3),
    ("f", 0, 1), ("f", 0, 2), ("s", 0, 4),
    ("f", 1, 1), ("f", 1, 2), ("s", 1, 4),
    ("f", 0, 3), ("f", 1, 3),
    ("f", 0, 4), ("s", 0, 5), ("f", 1, 4), ("s", 1, 5),
    ("f", 0, 5), ("s", 0, 6), ("s", 0, 8), ("s", 0, 11),
    ("f", 1, 5), ("s", 1, 6), ("s", 1, 8), ("s", 1, 11),
    ("f", 0, 6), ("s", 0, 7), ("s", 0, 9), ("s", 0, 12),
    ("f", 1, 6), ("s", 1, 7), ("s", 1, 9), ("s", 1, 12),
    ("f", 0, 7), ("s", 0, 10), ("s", 0, 13),
    ("f", 1, 7), ("s", 1, 10), ("s", 1, 13),
    ("w", 0, 8), ("s", 0, 14), ("p", 0, 8),
    ("w", 1, 8), ("s", 1, 14), ("p", 1, 8),
    ("w", 0, 9), ("s", 0, 15), ("p", 0, 9),
    ("w", 1, 9), ("s", 1, 15), ("p", 1, 9),
    ("w", 0, 10), ("s", 0, 16), ("p", 0, 10),
    ("w", 1, 10), ("s", 1, 16), ("p", 1, 10),
    ("f", 0, 11), ("f", 0, 12), ("f", 0, 13),
    ("f", 0, 14), ("f", 0, 15), ("f", 0, 16),
    ("f", 1, 11), ("f", 1, 12), ("f", 1, 13),
    ("f", 1, 14), ("f", 1, 15), ("f", 1, 16),
]


def kernel(x):
    def body(x_ref, out_ref, sb_a, rb_a, sb_b, rb_b, sems):
        me = lax.axis_index("i")
        p = lax.rem(me, 4)
        z = lax.div(me, 4)
        xb = jnp.where((p == 1) | (p == 2), 1, 0)
        yb = jnp.where(p >= 2, 1, 0)

        x_partner = z * 4 + jnp.where(lax.rem(p, 2) == 0, p + 1, p - 1)
        y_partner = z * 4 + (3 - p)
        zbit0 = lax.rem(z, 2)
        zbit1 = lax.div(z, 2)
        zp1 = (z + 1 - 2 * zbit0) * 4 + p
        zp2 = (z + 2 - 4 * zbit1) * 4 + p

        barrier_sem = pltpu.get_barrier_semaphore()
        for nbr in [x_partner, y_partner, zp1, zp2]:
            pl.semaphore_signal(
                barrier_sem,
                inc=1,
                device_id=(nbr,),
                device_id_type=pl.DeviceIdType.MESH,
            )
        pl.semaphore_wait(barrier_sem, 4)

        def half_units(c0, sel1, sel2, pn1, pn2, zorder, sb):
            mine_1 = sel1 * 1024
            theirs_1 = (1 - sel1) * 1024
            mine_2 = mine_1 + sel2 * 512
            theirs_2 = mine_1 + (1 - sel2) * 512
            send_s1 = theirs_1 + (1 - sel2) * 512
            send_s2 = theirs_1 + sel2 * 512
            recv_s2 = theirs_1 + (1 - sel2) * 512

            (zb_a, zp_a), (zb_b, zp_b) = zorder
            z_keep1 = mine_2 + zb_a * 256
            z_send1 = mine_2 + (1 - zb_a) * 256
            z_keep2 = z_keep1 + zb_b * 128
            z_send2 = z_keep1 + (1 - zb_b) * 128
            koff2 = zb_a * 256 + zb_b * 128
            soff2 = zb_a * 256 + (1 - zb_b) * 128
            soff1 = (1 - zb_a) * 256
            sra_off = (1 - zb_a) * 256
            srb_off = zb_a * 256

            def col(ref, base, rows):
                return ref[pl.ds(base, rows), pl.ds(c0, W)]

            def setcol(ref, base, rows, val):
                ref[pl.ds(base, rows), pl.ds(c0, W)] = val

            def sbput(base, rows, val):
                sb[pl.ds(base, rows), :] = val

            units = []

            units.append((
                pn1,
                lambda b: b.__setitem__(
                    ..., col(x_ref, send_s1, 512).astype(BF16)
                ),
                lambda rb: setcol(
                    out_ref, theirs_2, 512,
                    (col(x_ref, theirs_2, 512)
                     + rb[...].astype(F32)).astype(BF16),
                ),
            ))
            units.append((
                pn1,
                lambda b: b.__setitem__(
                    ..., col(x_ref, send_s2, 512).astype(BF16)
                ),
                lambda rb: setcol(
                    out_ref, mine_2, 512,
                    (col(x_ref, mine_2, 512)
                     + rb[...].astype(F32)).astype(BF16),
                ),
            ))
            units.append((
                pn2,
                lambda b: b.__setitem__(
                    ..., col(out_ref, theirs_2 + sra_off, 256)
                ),
                lambda rb: setcol(
                    out_ref, z_send1, 256,
                    col(out_ref, z_send1, 256) + rb[...],
                ),
            ))
            units.append((
                pn2,
                lambda b: b.__setitem__(
                    ..., col(out_ref, theirs_2 + srb_off, 256)
                ),
                lambda rb: setcol(
                    out_ref, z_keep1, 256,
                    col(out_ref, z_keep1, 256) + rb[...],
                ),
            ))
            units.append((
                zp_a,
                lambda b: b.__setitem__(..., col(out_ref, z_send1, 256)),
                lambda rb: setcol(
                    out_ref, z_keep1, 256,
                    col(out_ref, z_keep1, 256) + rb[...],
                ),
            ))

            def proc5(rb):
                vb = col(out_ref, z_keep2, 128) + rb[...]
                setcol(out_ref, z_keep2, 128, vb)
                sbput(OFFS[6], 128, vb)
                sbput(OFFS[7] + zb_b * 128, 128, vb)
                sbput(OFFS[8], 128, vb)
                sbput(OFFS[11], 128, vb)

            units.append((
                zp_b,
                lambda b: b.__setitem__(..., col(out_ref, z_send2, 128)),
                proc5,
            ))

            def proc6(rb):
                rbv = rb[...]
                setcol(out_ref, z_send2, 128, rbv)
                sbput(OFFS[7] + (1 - zb_b) * 128, 128, rbv)
                sbput(OFFS[9], 128, rbv)
                sbput(OFFS[12], 128, rbv)

            units.append((zp_b, None, proc6))

            def proc7(rb):
                rbv = rb[...]
                setcol(out_ref, z_send1, 256, rbv)
                sbput(OFFS[10], 256, rbv)
                sbput(OFFS[13], 256, rbv)

            units.append((zp_a, None, proc7))

            units.append((
                pn2,
                None,
                lambda rb: setcol(out_ref, theirs_2 + koff2, 128, rb[...]),
            ))
            units.append((
                pn2,
                None,
                lambda rb: setcol(out_ref, theirs_2 + soff2, 128, rb[...]),
            ))
            units.append((
                pn2,
                None,
                lambda rb: setcol(out_ref, theirs_2 + soff1, 256, rb[...]),
            ))
            units.append((
                pn1,
                None,
                lambda rb: setcol(out_ref, send_s2 + koff2, 128, rb[...]),
            ))
            units.append((
                pn1,
                None,
                lambda rb: setcol(out_ref, send_s2 + soff2, 128, rb[...]),
            ))
            units.append((
                pn1,
                None,
                lambda rb: setcol(out_ref, send_s2 + soff1, 256, rb[...]),
            ))
            units.append((
                pn1,
                None,
                lambda rb: setcol(out_ref, recv_s2 + koff2, 128, rb[...]),
            ))
            units.append((
                pn1,
                None,
                lambda rb: setcol(out_ref, recv_s2 + soff2, 128, rb[...]),
            ))
            units.append((
                pn1,
                None,
                lambda rb: setcol(out_ref, recv_s2 + soff1, 256, rb[...]),
            ))
            return units

        units_a = half_units(
            0, xb, yb, x_partner, y_partner,
            ((zbit0, zp1), (zbit1, zp2)), sb_a,
        )
        units_b = half_units(
            W, yb, xb, y_partner, x_partner,
            ((zbit1, zp2), (zbit0, zp1)), sb_b,
        )

        halves = [
            (units_a, sb_a, rb_a, 0, 1),
            (units_b, sb_b, rb_b, 2, 3),
        ]

        def make_rdma(h, k):
            units, sb, rb, srow, rrow = halves[h]
            partner = units[k][0]
            if k in FWD:
                j = FWD[k]
                src = rb.at[pl.ds(OFFS[j], ROWS[j]), :]
            else:
                src = sb.at[pl.ds(OFFS[k], ROWS[k]), :]
            return pltpu.make_async_remote_copy(
                src_ref=src,
                dst_ref=rb.at[pl.ds(OFFS[k], ROWS[k]), :],
                send_sem=sems.at[srow, k],
                recv_sem=sems.at[rrow, k],
                device_id=(partner,),
                device_id_type=pl.DeviceIdType.MESH,
            )

        for op, h, k in SCHED:
            units, sb, rb, _, _ = halves[h]
            _, prep, proc = units[k]
            if op == "s":
                if prep is not None:
                    prep(sb.at[pl.ds(OFFS[k], ROWS[k]), :])
                make_rdma(h, k).start()
            elif op == "w":
                make_rdma(h, k).wait_recv()
            elif op == "p":
                proc(rb.at[pl.ds(OFFS[k], ROWS[k]), :])
            else:
                make_rdma(h, k).wait_recv()
                proc(rb.at[pl.ds(OFFS[k], ROWS[k]), :])

        started = sorted({(h2, k2) for op2, h2, k2 in SCHED if op2 == "s"})
        recvd = {(h2, k2) for op2, h2, k2 in SCHED if op2 in ("f", "w")}
        for h2, k2 in started:
            if (h2, k2) not in recvd:
                make_rdma(h2, k2).wait_recv()
        for h2, k2 in started:
            make_rdma(h2, k2).wait_send()

    out_shape = jax.ShapeDtypeStruct((M, N), BF16)
    return pl.pallas_call(
        body,
        out_shape=out_shape,
        in_specs=[pl.BlockSpec(memory_space=pltpu.VMEM)],
        out_specs=pl.BlockSpec(memory_space=pltpu.VMEM),
        scratch_shapes=[
            pltpu.VMEM((TOT, W), BF16),
            pltpu.VMEM((TOT, W), BF16),
            pltpu.VMEM((TOT, W), BF16),
            pltpu.VMEM((TOT, W), BF16),
            pltpu.SemaphoreType.DMA((4, len(ROWS))),
        ],
        compiler_params=pltpu.CompilerParams(collective_id=0),
    )(x)
